# baseline (speedup 1.0000x reference)
import functools

import jax
import jax.numpy as jnp
from jax import lax
from jax.experimental import pallas as pl
from jax.experimental.pallas import tpu as pltpu
from jax.experimental.pallas import tpu_sc as plsc

_D = 64
_EPS = 1e-12
_CB = 8


def _sc_gather(token_table, idx_a, idx_b, b, s):
    n_rows = b * s
    sa = idx_a.shape[1]
    sb = s - sa
    info = plsc.get_sparse_core_info()
    nw = info.num_cores * info.num_subcores
    bpw = b // nw
    n_chunks = bpw // _CB
    mesh = plsc.VectorSubcoreMesh(core_axis_name="c", subcore_axis_name="s")

    @functools.partial(
        pl.kernel,
        mesh=mesh,
        compiler_params=pltpu.CompilerParams(use_tc_tiling_on_sc=False),
        out_type=jax.ShapeDtypeStruct((n_rows, _D), jnp.float32),
        scratch_types=[
            pltpu.VMEM((_CB, sa), jnp.int32),
            pltpu.VMEM((_CB, sa), jnp.int32),
            pltpu.VMEM((_CB * s, _D), jnp.float32),
            pltpu.SemaphoreType.DMA,
        ],
    )
    def k(table_hbm, ia_hbm, ib_hbm, out_hbm, ia_v, ib_v, rows_v, sem):
        cid = lax.axis_index("c")
        sid = lax.axis_index("s")
        wid = sid * info.num_cores + cid

        def chunk(g, carry):
            r0 = wid * bpw + g * _CB
            pltpu.sync_copy(ia_hbm.at[pl.ds(r0, _CB)], ia_v)
            pltpu.sync_copy(ib_hbm.at[pl.ds(r0, _CB)], ib_v)
            copies = []
            for i in range(_CB):
                copies.append(
                    pltpu.async_copy(
                        table_hbm.at[ia_v.at[i]],
                        rows_v.at[pl.ds(i * s, sa)],
                        sem,
                    )
                )
                copies.append(
                    pltpu.async_copy(
                        table_hbm.at[ib_v.at[i, pl.ds(0, sb)]],
                        rows_v.at[pl.ds(i * s + sa, sb)],
                        sem,
                    )
                )
            for c in copies:
                c.wait()
            pltpu.sync_copy(rows_v, out_hbm.at[pl.ds(r0 * s, _CB * s)])
            return carry

        lax.fori_loop(0, n_chunks, chunk, 0)

    return k(token_table, idx_a, idx_b)


def _tc_layernorm(gathered3d, pos3d, gamma3d, beta3d):
    b, s, d = gathered3d.shape
    bb = 32

    def body(x_ref, pos_ref, gamma_ref, beta_ref, o_ref):
        x = x_ref[...] + pos_ref[...]
        mean = jnp.mean(x, axis=-1, keepdims=True)
        xc = x - mean
        var = jnp.mean(xc * xc, axis=-1, keepdims=True)
        o_ref[...] = (
            xc * lax.rsqrt(var + _EPS) * gamma_ref[...] + beta_ref[...]
        )

    return pl.pallas_call(
        body,
        grid=(b // bb,),
        in_specs=[
            pl.BlockSpec((bb, s, d), lambda i: (i, 0, 0)),
            pl.BlockSpec((1, s, d), lambda i: (0, 0, 0)),
            pl.BlockSpec((1, 1, d), lambda i: (0, 0, 0)),
            pl.BlockSpec((1, 1, d), lambda i: (0, 0, 0)),
        ],
        out_specs=pl.BlockSpec((bb, s, d), lambda i: (i, 0, 0)),
        out_shape=jax.ShapeDtypeStruct((b, s, d), jnp.float32),
    )(gathered3d, pos3d, gamma3d, beta3d)


def _sc_gather_pairs(tt2, idx2d, n_rows):
    info = plsc.get_sparse_core_info()
    nw = info.num_cores * info.num_subcores
    per_w = n_rows // nw
    f, g = 1280, 128
    n_chunks = per_w // f
    mesh = plsc.VectorSubcoreMesh(core_axis_name="c", subcore_axis_name="s")

    @functools.partial(
        pl.kernel,
        mesh=mesh,
        compiler_params=pltpu.CompilerParams(use_tc_tiling_on_sc=False),
        out_type=jax.ShapeDtypeStruct((n_rows, _D), jnp.float32),
        scratch_types=[
            pltpu.VMEM((f // g, g), jnp.int32),
            pltpu.VMEM((f, _D), jnp.float32),
            pltpu.SemaphoreType.DMA,
        ],
    )
    def k(table_hbm, idx_hbm, out_hbm, idx_v, rows_v, sem):
        cid = lax.axis_index("c")
        sid = lax.axis_index("s")
        wid = sid * info.num_cores + cid

        def chunk(c, carry):
            base = wid * per_w + c * f
            pltpu.sync_copy(idx_hbm.at[pl.ds(base // g, f // g)], idx_v)
            copies = []
            for j in range(f // g):
                copies.append(
                    pltpu.async_copy(
                        table_hbm.at[idx_v.at[j], pl.ds(0, _D)],
                        rows_v.at[pl.ds(j * g, g)],
                        sem,
                    )
                )
            for cp in copies:
                cp.wait()
            pltpu.sync_copy(rows_v, out_hbm.at[pl.ds(base, f)])
            return carry

        lax.fori_loop(0, n_chunks, chunk, 0)

    return k(tt2, idx2d)


def kernel(input_ids, token_table, pos_table, gamma, beta):
    b, s = input_ids.shape
    v = token_table.shape[0]
    tt2 = token_table.reshape(v // 2, 2 * _D)
    tt3 = lax.optimization_barrier(tt2).reshape(v, _D)
    idx_a = input_ids[:, :128]
    idx_b = jnp.pad(input_ids[:, 128:], ((0, 0), (0, 128 - (s - 128))))
    gathered = _sc_gather(tt3, idx_a, idx_b, b, s)
    return _tc_layernorm(
        gathered.reshape(b, s, _D),
        pos_table.reshape(1, s, _D),
        gamma.reshape(1, 1, _D),
        beta.reshape(1, 1, _D),
    )


# speedup vs baseline: 1.0035x; 1.0035x over previous
import functools

import jax
import jax.numpy as jnp
from jax import lax
from jax.experimental import pallas as pl
from jax.experimental.pallas import tpu as pltpu
from jax.experimental.pallas import tpu_sc as plsc

_D = 64
_EPS = 1e-12
_CB = 8


def _sc_gather(token_table, idx_a, idx_b, b, s):
    n_rows = b * s
    sa = idx_a.shape[1]
    sb = s - sa
    info = plsc.get_sparse_core_info()
    nw = info.num_cores * info.num_subcores
    bpw = b // nw
    n_chunks = bpw // _CB
    mesh = plsc.VectorSubcoreMesh(core_axis_name="c", subcore_axis_name="s")

    @functools.partial(
        pl.kernel,
        mesh=mesh,
        compiler_params=pltpu.CompilerParams(use_tc_tiling_on_sc=False),
        out_type=jax.ShapeDtypeStruct((n_rows, _D), jnp.float32),
        scratch_types=[
            pltpu.VMEM((_CB, sa), jnp.int32),
            pltpu.VMEM((_CB, sa), jnp.int32),
            pltpu.VMEM((_CB * s, _D), jnp.float32),
            pltpu.SemaphoreType.DMA,
        ],
    )
    def k(table_hbm, ia_hbm, ib_hbm, out_hbm, ia_v, ib_v, rows_v, sem):
        cid = lax.axis_index("c")
        sid = lax.axis_index("s")
        wid = sid * info.num_cores + cid

        def chunk(g, carry):
            r0 = wid * bpw + g * _CB
            pltpu.sync_copy(ia_hbm.at[pl.ds(r0, _CB)], ia_v)
            pltpu.sync_copy(ib_hbm.at[pl.ds(r0, _CB)], ib_v)
            copies = []
            for i in range(_CB):
                copies.append(
                    pltpu.async_copy(
                        table_hbm.at[ia_v.at[i]],
                        rows_v.at[pl.ds(i * s, sa)],
                        sem,
                    )
                )
                copies.append(
                    pltpu.async_copy(
                        table_hbm.at[ib_v.at[i, pl.ds(0, sb)]],
                        rows_v.at[pl.ds(i * s + sa, sb)],
                        sem,
                    )
                )
            for c in copies:
                c.wait()
            pltpu.sync_copy(rows_v, out_hbm.at[pl.ds(r0 * s, _CB * s)])
            return carry

        lax.fori_loop(0, n_chunks, chunk, 0)

    return k(token_table, idx_a, idx_b)


def _tc_layernorm(gathered3d, pos3d, gamma3d, beta3d):
    b, s, d = gathered3d.shape
    bb = 32

    def body(x_ref, pos_ref, gamma_ref, beta_ref, o_ref):
        x = x_ref[...] + pos_ref[...]
        mean = jnp.mean(x, axis=-1, keepdims=True)
        xc = x - mean
        var = jnp.mean(xc * xc, axis=-1, keepdims=True)
        o_ref[...] = (
            xc * lax.rsqrt(var + _EPS) * gamma_ref[...] + beta_ref[...]
        )

    return pl.pallas_call(
        body,
        grid=(b // bb,),
        in_specs=[
            pl.BlockSpec((bb, s, d), lambda i: (i, 0, 0)),
            pl.BlockSpec((1, s, d), lambda i: (0, 0, 0)),
            pl.BlockSpec((1, 1, d), lambda i: (0, 0, 0)),
            pl.BlockSpec((1, 1, d), lambda i: (0, 0, 0)),
        ],
        out_specs=pl.BlockSpec((bb, s, d), lambda i: (i, 0, 0)),
        out_shape=jax.ShapeDtypeStruct((b, s, d), jnp.float32),
    )(gathered3d, pos3d, gamma3d, beta3d)


def kernel(input_ids, token_table, pos_table, gamma, beta):
    b, s = input_ids.shape
    idx_a = input_ids[:, :128]
    idx_b = jnp.pad(input_ids[:, 128:], ((0, 0), (0, 128 - (s - 128))))
    gathered = _sc_gather(token_table, idx_a, idx_b, b, s)
    return _tc_layernorm(
        gathered.reshape(b, s, _D),
        pos_table.reshape(1, s, _D),
        gamma.reshape(1, 1, _D),
        beta.reshape(1, 1, _D),
    )
